# baseline (speedup 1.0000x reference)
import functools

import jax
import jax.numpy as jnp
from jax import lax
from jax.experimental import pallas as pl
from jax.experimental.pallas import tpu as pltpu
from jax.experimental.pallas import tpu_sc as plsc

B = 16384
D = 64
L = 16

_NC = 2
_NS = 16
_NW = _NC * _NS
_BPW = B // _NW
_NG = _BPW // L

_mesh = plsc.VectorSubcoreMesh(core_axis_name="c", subcore_axis_name="s")


@functools.partial(
    pl.kernel,
    out_type=jax.ShapeDtypeStruct((B,), jnp.float32),
    mesh=_mesh,
    scratch_types=[
        pltpu.VMEM((_BPW,), jnp.int32),
        pltpu.VMEM((_BPW,), jnp.int32),
        pltpu.VMEM((_BPW, D), jnp.float32),
        pltpu.VMEM((_BPW, D), jnp.float32),
        pltpu.VMEM((_BPW,), jnp.float32),
        pltpu.VMEM((_BPW,), jnp.float32),
        pltpu.VMEM((L,), jnp.float32),
        pltpu.VMEM((_BPW,), jnp.float32),
        pltpu.VMEM((L * (L + 1),), jnp.float32),
        pltpu.SemaphoreType.DMA,
    ],
    compiler_params=pltpu.CompilerParams(needs_layout_passes=False,
                                         use_tc_tiling_on_sc=False),
)
def _mf_kernel(u_id, i_id, user_emb, user_bias, item_emb, item_bias, mean,
               out, u_idx_v, i_idx_v, u_rows_v, i_rows_v, bu_v, bi_v,
               mean_v, out_v, tr_v, sem):
  wid = lax.axis_index("s") * _NC + lax.axis_index("c")
  base = wid * _BPW

  pltpu.sync_copy(u_id.at[pl.ds(base, _BPW)], u_idx_v)
  pltpu.sync_copy(i_id.at[pl.ds(base, _BPW)], i_idx_v)
  pltpu.sync_copy(mean, mean_v)

  c1 = pltpu.async_copy(user_emb.at[u_idx_v], u_rows_v, sem)
  c2 = pltpu.async_copy(item_emb.at[i_idx_v], i_rows_v, sem)
  c3 = pltpu.async_copy(user_bias.at[u_idx_v], bu_v, sem)
  c4 = pltpu.async_copy(item_bias.at[i_idx_v], bi_v, sem)
  c1.wait()
  c2.wait()
  c3.wait()
  c4.wait()

  mvec = mean_v[...]
  lane17 = lax.iota(jnp.int32, L) * (L + 1)

  def group(g, carry):
    gb = g * L
    for r in range(L):
      b = gb + r
      acc = u_rows_v[b, pl.ds(0, L)] * i_rows_v[b, pl.ds(0, L)]
      for j in range(1, D // L):
        acc = acc + (u_rows_v[b, pl.ds(j * L, L)] *
                     i_rows_v[b, pl.ds(j * L, L)])
      tr_v[pl.ds(r * (L + 1), L)] = acc
    dots = mvec
    for c in range(L):
      dots = dots + plsc.load_gather(tr_v, [lane17 + c])
    out_v[pl.ds(gb, L)] = dots + bu_v[pl.ds(gb, L)] + bi_v[pl.ds(gb, L)]
    return carry

  lax.fori_loop(0, _NG, group, 0)

  pltpu.sync_copy(out_v, out.at[pl.ds(base, _BPW)])


def kernel(u_id, i_id, user_emb, user_bias, item_emb, item_bias, mean):
  u_id = u_id.astype(jnp.int32)
  i_id = i_id.astype(jnp.int32)
  mean16 = jnp.broadcast_to(mean, (L,))
  return _mf_kernel(u_id, i_id, user_emb, jnp.reshape(user_bias, (-1,)),
                    item_emb, jnp.reshape(item_bias, (-1,)), mean16)


# speedup vs baseline: 2.1873x; 2.1873x over previous
import functools

import jax
import jax.numpy as jnp
from jax import lax
from jax.experimental import pallas as pl
from jax.experimental.pallas import tpu as pltpu
from jax.experimental.pallas import tpu_sc as plsc

B = 16384
D = 64
L = 16
SL = 8

_NC = 2
_NS = 16
_NW = _NC * _NS
_BPW = B // _NW
_C = 16
_NCH = _BPW // _C

_mesh = plsc.VectorSubcoreMesh(core_axis_name="c", subcore_axis_name="s")


@functools.partial(
    pl.kernel,
    out_type=jax.ShapeDtypeStruct((B,), jnp.float32),
    mesh=_mesh,
    scratch_types=[
        pltpu.VMEM((_BPW,), jnp.int32),
        pltpu.VMEM((_BPW,), jnp.int32),
        pltpu.VMEM((2, _C, SL, D), jnp.float32),
        pltpu.VMEM((2, _C, SL, D), jnp.float32),
        pltpu.VMEM((_BPW,), jnp.float32),
        pltpu.VMEM((_BPW,), jnp.float32),
        pltpu.VMEM((L,), jnp.float32),
        pltpu.VMEM((_BPW,), jnp.float32),
        pltpu.VMEM((L * (L + 1),), jnp.float32),
        pltpu.SemaphoreType.DMA,
        pltpu.SemaphoreType.DMA,
        pltpu.SemaphoreType.DMA,
    ],
    compiler_params=pltpu.CompilerParams(needs_layout_passes=False),
)
def _mf_kernel(u_id, i_id, user_emb, user_bias, item_emb, item_bias, mean,
               out, u_idx_v, i_idx_v,
               u_blk, i_blk, bu_v, bi_v, mean_v, out_v, tr_v,
               sem0, sem1, semb):
  wid = lax.axis_index("s") * _NC + lax.axis_index("c")
  base = wid * _BPW
  sems = (sem0, sem1)

  pltpu.sync_copy(u_id.at[pl.ds(base, _BPW)], u_idx_v)
  pltpu.sync_copy(i_id.at[pl.ds(base, _BPW)], i_idx_v)

  cb1 = pltpu.async_copy(user_bias.at[u_idx_v], bu_v, semb)
  cb2 = pltpu.async_copy(item_bias.at[i_idx_v], bi_v, semb)

  pltpu.sync_copy(mean, mean_v)

  def fire(ch, slot):
    uvec = u_idx_v[pl.ds(ch * _C, _C)]
    ivec = i_idx_v[pl.ds(ch * _C, _C)]
    for r in range(_C):
      gu = uvec[r] >> 3
      gi = ivec[r] >> 3
      pltpu.async_copy(user_emb.at[gu], u_blk.at[slot, r], sems[slot])
      pltpu.async_copy(item_emb.at[gi], i_blk.at[slot, r], sems[slot])

  def drain(slot):
    pltpu.make_async_copy(user_emb.at[pl.ds(0, _C)],
                          u_blk.at[slot], sems[slot]).wait()
    pltpu.make_async_copy(item_emb.at[pl.ds(0, _C)],
                          i_blk.at[slot], sems[slot]).wait()

  cb1.wait()
  cb2.wait()
  mvec = mean_v[...]
  lane17 = lax.iota(jnp.int32, L) * (L + 1)

  fire(0, 0)

  def compute(ch, slot):
    uvec = u_idx_v[pl.ds(ch * _C, _C)]
    ivec = i_idx_v[pl.ds(ch * _C, _C)]
    for r in range(_C):
      su = uvec[r] & 7
      si = ivec[r] & 7
      acc = (u_blk[slot, r, su, pl.ds(0, L)] *
             i_blk[slot, r, si, pl.ds(0, L)])
      for j in range(1, D // L):
        acc = acc + (u_blk[slot, r, su, pl.ds(j * L, L)] *
                     i_blk[slot, r, si, pl.ds(j * L, L)])
      tr_v[pl.ds(r * (L + 1), L)] = acc
    dots = mvec
    for c in range(L):
      dots = dots + plsc.load_gather(tr_v, [lane17 + c])
    sl = pl.ds(ch * _C, L)
    out_v[sl] = dots + bu_v[sl] + bi_v[sl]

  def step(t, carry):
    ch0 = t * 2
    fire(ch0 + 1, 1)
    drain(0)
    compute(ch0, 0)

    @pl.when(ch0 + 2 < _NCH)
    def _():
      fire(ch0 + 2, 0)

    drain(1)
    compute(ch0 + 1, 1)
    return carry

  lax.fori_loop(0, _NCH // 2, step, 0)

  pltpu.sync_copy(out_v, out.at[pl.ds(base, _BPW)])


def kernel(u_id, i_id, user_emb, user_bias, item_emb, item_bias, mean):
  u_id = u_id.astype(jnp.int32)
  i_id = i_id.astype(jnp.int32)
  mean16 = jnp.broadcast_to(mean, (L,))
  u3 = jnp.reshape(user_emb, (user_emb.shape[0] // SL, SL, D))
  i3 = jnp.reshape(item_emb, (item_emb.shape[0] // SL, SL, D))
  return _mf_kernel(u_id, i_id, u3, jnp.reshape(user_bias, (-1,)),
                    i3, jnp.reshape(item_bias, (-1,)), mean16)
